# baseline (speedup 1.0000x reference)
import functools

import jax
import jax.numpy as jnp
from jax import lax
from jax.experimental import pallas as pl
from jax.experimental.pallas import tpu as pltpu
from jax.experimental.pallas import tpu_sc as plsc

N = 10000
H = 128
E = 320000
TIMESTEPS = 4

NC = 2
NS = 16
NW = NC * NS
CHUNK = 128
HC = CHUNK // 2
CH_PER_TILE = 80
BLK = 40
NBLK = CH_PER_TILE // BLK
E_PAD = NW * CH_PER_TILE * CHUNK
N_PAD = 10240
ROWS_PER_TILE = N_PAD // NS

RB = 200
GRID = N // RB


def _edge_loop(prop_hbm, src_hbm, tgt_hbm, zr_hbm, out_hbm, srca, tgta,
               rows0, rows1, acc, g0, g1, s0, s1, extra=None):
    c = lax.axis_index("c")
    s = lax.axis_index("s")
    wid = s * NC + c
    row0 = s * ROWS_PER_TILE
    pltpu.sync_copy(zr_hbm, acc.at[pl.ds(row0, ROWS_PER_TILE)])
    if extra is not None:
        z1_hbm, ones_hbm, cout_hbm, ones, acc1, csem = extra
        pltpu.sync_copy(z1_hbm, acc1.at[pl.ds(row0, ROWS_PER_TILE)])
        pltpu.sync_copy(ones_hbm, ones)
    plsc.subcore_barrier()

    bufs = [(rows0, g0, s0), (rows1, g1, s1)]

    for blk in range(NBLK):
        pltpu.sync_copy(src_hbm.at[wid, pl.ds(blk * BLK, BLK)], srca)
        pltpu.sync_copy(tgt_hbm.at[wid, pl.ds(blk * BLK, BLK)], tgta)
        pltpu.async_copy(prop_hbm.at[srca.at[0]], rows0, g0)
        for k in range(BLK):
            rb, gs, ss = bufs[k % 2]
            pltpu.make_async_copy(prop_hbm.at[srca.at[k]], rb, gs).wait()
            if k + 1 < BLK:
                ob, og, osem = bufs[(k + 1) % 2]
                if k >= 1:
                    pltpu.make_async_copy(ob, acc.at[tgta.at[k - 1]],
                                          osem).wait()
                pltpu.async_copy(prop_hbm.at[srca.at[k + 1]], ob, og)
            pltpu.async_copy(rb, acc.at[tgta.at[k]], ss, add=True)
            if extra is not None:
                pltpu.async_copy(ones, acc1.at[tgta.at[k]], csem, add=True)
        rb, _, ss = bufs[(BLK - 2) % 2]
        pltpu.make_async_copy(rb, acc.at[tgta.at[BLK - 2]], ss).wait()
        rb, _, ss = bufs[(BLK - 1) % 2]
        pltpu.make_async_copy(rb, acc.at[tgta.at[BLK - 1]], ss).wait()
        if extra is not None:
            for k in range(BLK):
                pltpu.make_async_copy(ones, acc1.at[tgta.at[k]], csem).wait()

    plsc.subcore_barrier()
    pltpu.sync_copy(acc.at[pl.ds(row0, ROWS_PER_TILE)],
                    out_hbm.at[c, pl.ds(row0, ROWS_PER_TILE)])
    if extra is not None:
        pltpu.sync_copy(acc1.at[pl.ds(row0, ROWS_PER_TILE)],
                        cout_hbm.at[c, pl.ds(row0, ROWS_PER_TILE)])


def _edge_body(prop_hbm, src_hbm, tgt_hbm, zr_hbm, out_hbm, srca, tgta,
               rows0, rows1, acc, g0, g1, s0, s1):
    _edge_loop(prop_hbm, src_hbm, tgt_hbm, zr_hbm, out_hbm, srca, tgta,
               rows0, rows1, acc, g0, g1, s0, s1)


def _edge_cnt_body(prop_hbm, src_hbm, tgt_hbm, zr_hbm, z1_hbm, ones_hbm,
                   out_hbm, cout_hbm, srca, tgta, rows0, rows1, acc, ones,
                   acc1, g0, g1, s0, s1, csem):
    _edge_loop(prop_hbm, src_hbm, tgt_hbm, zr_hbm, out_hbm, srca, tgta,
               rows0, rows1, acc, g0, g1, s0, s1,
               extra=(z1_hbm, ones_hbm, cout_hbm, ones, acc1, csem))


@functools.lru_cache(maxsize=None)
def _sc_kernels():
    mesh = plsc.VectorSubcoreMesh(core_axis_name="c", subcore_axis_name="s",
                                  num_cores=NC, num_subcores=NS)
    edge = pl.kernel(
        _edge_body,
        out_type=jax.ShapeDtypeStruct((NC, N_PAD, H), jnp.float32),
        mesh=mesh,
        scratch_types=[
            pltpu.VMEM((BLK, CHUNK), jnp.int32),
            pltpu.VMEM((BLK, CHUNK), jnp.int32),
            pltpu.VMEM((CHUNK, H), jnp.float32),
            pltpu.VMEM((CHUNK, H), jnp.float32),
            pltpu.VMEM_SHARED((N_PAD, H), jnp.float32),
            pltpu.SemaphoreType.DMA,
            pltpu.SemaphoreType.DMA,
            pltpu.SemaphoreType.DMA,
            pltpu.SemaphoreType.DMA,
        ],
    )
    edge_cnt = pl.kernel(
        _edge_cnt_body,
        out_type=[jax.ShapeDtypeStruct((NC, N_PAD, H), jnp.float32),
                  jax.ShapeDtypeStruct((NC, N_PAD), jnp.float32)],
        mesh=mesh,
        scratch_types=[
            pltpu.VMEM((BLK, CHUNK), jnp.int32),
            pltpu.VMEM((BLK, CHUNK), jnp.int32),
            pltpu.VMEM((CHUNK, H), jnp.float32),
            pltpu.VMEM((CHUNK, H), jnp.float32),
            pltpu.VMEM_SHARED((N_PAD, H), jnp.float32),
            pltpu.VMEM((CHUNK,), jnp.float32),
            pltpu.VMEM_SHARED((N_PAD,), jnp.float32),
            pltpu.SemaphoreType.DMA,
            pltpu.SemaphoreType.DMA,
            pltpu.SemaphoreType.DMA,
            pltpu.SemaphoreType.DMA,
            pltpu.SemaphoreType.DMA,
        ],
    )
    return edge, edge_cnt


def _gru_math(p_ref, cnt_ref, h, wT_ih_ref, bih_ref, wT_hh_ref, bhh_ref):
    p = p_ref[0] + p_ref[1]
    cnt = cnt_ref[0] + cnt_ref[1]
    x = p / jnp.maximum(cnt, 1.0)
    gi = jnp.dot(x, wT_ih_ref[...], preferred_element_type=jnp.float32)
    gi = gi + bih_ref[0:1, :]
    gh = jnp.dot(h, wT_hh_ref[...], preferred_element_type=jnp.float32)
    gh = gh + bhh_ref[0:1, :]
    r = jax.nn.sigmoid(gi[:, :H] + gh[:, :H])
    z = jax.nn.sigmoid(gi[:, H:2 * H] + gh[:, H:2 * H])
    n = jnp.tanh(gi[:, 2 * H:] + r * gh[:, 2 * H:])
    return (1.0 - z) * n + z * h


def _mm0_body(h_ref, wT_msg_ref, bmsg_ref, prop_ref):
    y = jnp.dot(h_ref[...], wT_msg_ref[...],
                preferred_element_type=jnp.float32)
    prop_ref[...] = y + bmsg_ref[0:1, :]


def _step_body(p_ref, cnt_ref, h_ref, wT_ih_ref, bih_ref, wT_hh_ref,
               bhh_ref, wT_msg_ref, bmsg_ref, hn_ref, prop_ref):
    hn = _gru_math(p_ref, cnt_ref, h_ref[...], wT_ih_ref, bih_ref,
                   wT_hh_ref, bhh_ref)
    hn_ref[...] = hn
    y = jnp.dot(hn, wT_msg_ref[...], preferred_element_type=jnp.float32)
    prop_ref[...] = y + bmsg_ref[0:1, :]


def _last_body(p_ref, cnt_ref, h_ref, wT_ih_ref, bih_ref, wT_hh_ref,
               bhh_ref, hn_ref):
    hn_ref[...] = _gru_math(p_ref, cnt_ref, h_ref[...], wT_ih_ref, bih_ref,
                            wT_hh_ref, bhh_ref)


_spec_part = pl.BlockSpec((NC, RB, H), lambda i: (0, i, 0))
_spec_cnt = pl.BlockSpec((NC, RB, 1), lambda i: (0, i, 0))
_spec_h = pl.BlockSpec((RB, H), lambda i: (i, 0))
_spec_w3 = pl.BlockSpec((H, 3 * H), lambda i: (0, 0))
_spec_b3 = pl.BlockSpec((8, 3 * H), lambda i: (0, 0))
_spec_w1 = pl.BlockSpec((H, H), lambda i: (0, 0))
_spec_b1 = pl.BlockSpec((8, H), lambda i: (0, 0))

_sds_h = jax.ShapeDtypeStruct((N, H), jnp.float32)

_mm0 = pl.pallas_call(
    _mm0_body,
    grid=(GRID,),
    in_specs=[_spec_h, _spec_w1, _spec_b1],
    out_specs=_spec_h,
    out_shape=_sds_h,
)

_step = pl.pallas_call(
    _step_body,
    grid=(GRID,),
    in_specs=[_spec_part, _spec_cnt, _spec_h, _spec_w3, _spec_b3, _spec_w3,
              _spec_b3, _spec_w1, _spec_b1],
    out_specs=[_spec_h, _spec_h],
    out_shape=[_sds_h, _sds_h],
)

_last = pl.pallas_call(
    _last_body,
    grid=(GRID,),
    in_specs=[_spec_part, _spec_cnt, _spec_h, _spec_w3, _spec_b3, _spec_w3,
              _spec_b3],
    out_specs=_spec_h,
    out_shape=_sds_h,
)


def kernel(node_states, edge_lists, W_msg, b_msg, w_ih, w_hh, b_ih, b_hh):
    f32 = jnp.float32
    el = edge_lists[0]
    src = el[:, 0].astype(jnp.int32)
    tgt = el[:, 1].astype(jnp.int32)
    pad = E_PAD - E
    pad_src = jnp.arange(pad, dtype=jnp.int32) % N
    pad_tgt = N + (jnp.arange(pad, dtype=jnp.int32) % (N_PAD - N))
    srcp = jnp.concatenate([src, pad_src]).reshape(NW, CH_PER_TILE, CHUNK)
    tgtp = jnp.concatenate([tgt, pad_tgt]).reshape(NW, CH_PER_TILE, CHUNK)

    zrows = jnp.zeros((ROWS_PER_TILE, H), f32)
    z1 = jnp.zeros((ROWS_PER_TILE,), f32)
    ones1 = jnp.ones((CHUNK,), f32)

    edge_k, edge_cnt_k = _sc_kernels()

    wT_msg = W_msg.T
    bmsg = jnp.broadcast_to(b_msg[None, :], (8, H))
    wT_ih = w_ih.T
    bih = jnp.broadcast_to(b_ih[None, :], (8, 3 * H))
    wT_hh = w_hh.T
    bhh = jnp.broadcast_to(b_hh[None, :], (8, 3 * H))

    h = node_states
    prop = _mm0(h, wT_msg, bmsg)
    cnt_part = None
    for t in range(TIMESTEPS):
        if t == 0:
            part, cnt_raw = edge_cnt_k(prop, srcp, tgtp, zrows, z1, ones1)
            cnt_part = cnt_raw.reshape(NC, N_PAD, 1)
        else:
            part = edge_k(prop, srcp, tgtp, zrows)
        if t + 1 < TIMESTEPS:
            h, prop = _step(part, cnt_part, h, wT_ih, bih, wT_hh, bhh,
                            wT_msg, bmsg)
        else:
            h = _last(part, cnt_part, h, wT_ih, bih, wT_hh, bhh)
    return h, node_states


# speedup vs baseline: 1.0244x; 1.0244x over previous
import functools

import jax
import jax.numpy as jnp
from jax import lax
from jax.experimental import pallas as pl
from jax.experimental.pallas import tpu as pltpu
from jax.experimental.pallas import tpu_sc as plsc

N = 10000
H = 128
E = 320000
TIMESTEPS = 4

NC = 2
NS = 16
NW = NC * NS
CHUNK = 128
CH_PER_TILE = 80
BLK = 40
NBLK = CH_PER_TILE // BLK
E_PAD = NW * CH_PER_TILE * CHUNK
N_PAD = 10240
ROWS_PER_TILE = N_PAD // NS

RB = 200
GRID = N // RB


def _edge_body(prop_hbm, src_hbm, tgt_hbm, zr_hbm, out_hbm, srca, tgta,
               rows0, rows1, acc, g0, g1, s0, s1):
    c = lax.axis_index("c")
    s = lax.axis_index("s")
    wid = s * NC + c
    row0 = s * ROWS_PER_TILE
    pltpu.sync_copy(zr_hbm, acc.at[pl.ds(row0, ROWS_PER_TILE)])
    plsc.subcore_barrier()

    bufs = [(rows0, g0, s0), (rows1, g1, s1)]

    @pl.loop(0, NBLK)
    def _blk(blk):
        pltpu.sync_copy(src_hbm.at[wid, pl.ds(blk * BLK, BLK)], srca)
        pltpu.sync_copy(tgt_hbm.at[wid, pl.ds(blk * BLK, BLK)], tgta)
        pltpu.async_copy(prop_hbm.at[srca.at[0]], rows0, g0)
        for k in range(BLK):
            rb, gs, ss = bufs[k % 2]
            pltpu.make_async_copy(prop_hbm.at[srca.at[k]], rb, gs).wait()
            pltpu.async_copy(rb, acc.at[tgta.at[k]], ss, add=True)
            if k + 1 < BLK:
                ob, og, osem = bufs[(k + 1) % 2]
                if k >= 1:
                    pltpu.make_async_copy(ob, acc.at[tgta.at[k - 1]],
                                          osem).wait()
                pltpu.async_copy(prop_hbm.at[srca.at[k + 1]], ob, og)
        rb, _, ss = bufs[(BLK - 2) % 2]
        pltpu.make_async_copy(rb, acc.at[tgta.at[BLK - 2]], ss).wait()
        rb, _, ss = bufs[(BLK - 1) % 2]
        pltpu.make_async_copy(rb, acc.at[tgta.at[BLK - 1]], ss).wait()

    plsc.subcore_barrier()
    pltpu.sync_copy(acc.at[pl.ds(row0, ROWS_PER_TILE)],
                    out_hbm.at[c, pl.ds(row0, ROWS_PER_TILE)])


def _cnt_body(tgt_hbm, ones_hbm, z1_hbm, out_hbm, tgta, ones, acc1, csem):
    c = lax.axis_index("c")
    s = lax.axis_index("s")
    wid = s * NC + c
    row0 = s * ROWS_PER_TILE
    pltpu.sync_copy(z1_hbm, acc1.at[pl.ds(row0, ROWS_PER_TILE)])
    pltpu.sync_copy(tgt_hbm.at[wid], tgta)
    pltpu.sync_copy(ones_hbm, ones)
    plsc.subcore_barrier()

    for i in range(CH_PER_TILE):
        pltpu.async_copy(ones, acc1.at[tgta.at[i]], csem, add=True)
    for i in range(CH_PER_TILE):
        pltpu.make_async_copy(ones, acc1.at[tgta.at[i]], csem).wait()
    plsc.subcore_barrier()
    pltpu.sync_copy(acc1.at[pl.ds(row0, ROWS_PER_TILE)],
                    out_hbm.at[c, pl.ds(row0, ROWS_PER_TILE)])


@functools.lru_cache(maxsize=None)
def _sc_kernels():
    mesh = plsc.VectorSubcoreMesh(core_axis_name="c", subcore_axis_name="s",
                                  num_cores=NC, num_subcores=NS)
    edge = pl.kernel(
        _edge_body,
        out_type=jax.ShapeDtypeStruct((NC, N_PAD, H), jnp.float32),
        mesh=mesh,
        scratch_types=[
            pltpu.VMEM((BLK, CHUNK), jnp.int32),
            pltpu.VMEM((BLK, CHUNK), jnp.int32),
            pltpu.VMEM((CHUNK, H), jnp.float32),
            pltpu.VMEM((CHUNK, H), jnp.float32),
            pltpu.VMEM_SHARED((N_PAD, H), jnp.float32),
            pltpu.SemaphoreType.DMA,
            pltpu.SemaphoreType.DMA,
            pltpu.SemaphoreType.DMA,
            pltpu.SemaphoreType.DMA,
        ],
    )
    cnt = pl.kernel(
        _cnt_body,
        out_type=jax.ShapeDtypeStruct((NC, N_PAD), jnp.float32),
        mesh=mesh,
        scratch_types=[
            pltpu.VMEM((CH_PER_TILE, CHUNK), jnp.int32),
            pltpu.VMEM((CHUNK,), jnp.float32),
            pltpu.VMEM_SHARED((N_PAD,), jnp.float32),
            pltpu.SemaphoreType.DMA,
        ],
    )
    return edge, cnt


def _gru_math(p_ref, cnt_ref, h, wT_ih_ref, bih_ref, wT_hh_ref, bhh_ref):
    p = p_ref[0] + p_ref[1]
    cnt = cnt_ref[0] + cnt_ref[1]
    x = p / jnp.maximum(cnt, 1.0)
    gi = jnp.dot(x, wT_ih_ref[...], preferred_element_type=jnp.float32)
    gi = gi + bih_ref[0:1, :]
    gh = jnp.dot(h, wT_hh_ref[...], preferred_element_type=jnp.float32)
    gh = gh + bhh_ref[0:1, :]
    r = jax.nn.sigmoid(gi[:, :H] + gh[:, :H])
    z = jax.nn.sigmoid(gi[:, H:2 * H] + gh[:, H:2 * H])
    n = jnp.tanh(gi[:, 2 * H:] + r * gh[:, 2 * H:])
    return (1.0 - z) * n + z * h


def _mm0_body(h_ref, wT_msg_ref, bmsg_ref, prop_ref):
    y = jnp.dot(h_ref[...], wT_msg_ref[...],
                preferred_element_type=jnp.float32)
    prop_ref[...] = y + bmsg_ref[0:1, :]


def _step_body(p_ref, cnt_ref, h_ref, wT_ih_ref, bih_ref, wT_hh_ref,
               bhh_ref, wT_msg_ref, bmsg_ref, hn_ref, prop_ref):
    hn = _gru_math(p_ref, cnt_ref, h_ref[...], wT_ih_ref, bih_ref,
                   wT_hh_ref, bhh_ref)
    hn_ref[...] = hn
    y = jnp.dot(hn, wT_msg_ref[...], preferred_element_type=jnp.float32)
    prop_ref[...] = y + bmsg_ref[0:1, :]


def _last_body(p_ref, cnt_ref, h_ref, wT_ih_ref, bih_ref, wT_hh_ref,
               bhh_ref, hn_ref):
    hn_ref[...] = _gru_math(p_ref, cnt_ref, h_ref[...], wT_ih_ref, bih_ref,
                            wT_hh_ref, bhh_ref)


_spec_part = pl.BlockSpec((NC, RB, H), lambda i: (0, i, 0))
_spec_cnt = pl.BlockSpec((NC, RB, 1), lambda i: (0, i, 0))
_spec_h = pl.BlockSpec((RB, H), lambda i: (i, 0))
_spec_w3 = pl.BlockSpec((H, 3 * H), lambda i: (0, 0))
_spec_b3 = pl.BlockSpec((8, 3 * H), lambda i: (0, 0))
_spec_w1 = pl.BlockSpec((H, H), lambda i: (0, 0))
_spec_b1 = pl.BlockSpec((8, H), lambda i: (0, 0))

_sds_h = jax.ShapeDtypeStruct((N, H), jnp.float32)

_mm0 = pl.pallas_call(
    _mm0_body,
    grid=(GRID,),
    in_specs=[_spec_h, _spec_w1, _spec_b1],
    out_specs=_spec_h,
    out_shape=_sds_h,
)

_step = pl.pallas_call(
    _step_body,
    grid=(GRID,),
    in_specs=[_spec_part, _spec_cnt, _spec_h, _spec_w3, _spec_b3, _spec_w3,
              _spec_b3, _spec_w1, _spec_b1],
    out_specs=[_spec_h, _spec_h],
    out_shape=[_sds_h, _sds_h],
)

_last = pl.pallas_call(
    _last_body,
    grid=(GRID,),
    in_specs=[_spec_part, _spec_cnt, _spec_h, _spec_w3, _spec_b3, _spec_w3,
              _spec_b3],
    out_specs=_spec_h,
    out_shape=_sds_h,
)


def kernel(node_states, edge_lists, W_msg, b_msg, w_ih, w_hh, b_ih, b_hh):
    f32 = jnp.float32
    el = edge_lists[0]
    src = el[:, 0].astype(jnp.int32)
    tgt = el[:, 1].astype(jnp.int32)
    pad = E_PAD - E
    pad_src = jnp.arange(pad, dtype=jnp.int32) % N
    pad_tgt = N + (jnp.arange(pad, dtype=jnp.int32) % (N_PAD - N))
    srcp = jnp.concatenate([src, pad_src]).reshape(NW, CH_PER_TILE, CHUNK)
    tgtp = jnp.concatenate([tgt, pad_tgt]).reshape(NW, CH_PER_TILE, CHUNK)

    zrows = jnp.zeros((ROWS_PER_TILE, H), f32)
    z1 = jnp.zeros((ROWS_PER_TILE,), f32)
    ones1 = jnp.ones((CHUNK,), f32)

    edge_k, cnt_k = _sc_kernels()
    cnt_part = cnt_k(tgtp, ones1, z1).reshape(NC, N_PAD, 1)

    wT_msg = W_msg.T
    bmsg = jnp.broadcast_to(b_msg[None, :], (8, H))
    wT_ih = w_ih.T
    bih = jnp.broadcast_to(b_ih[None, :], (8, 3 * H))
    wT_hh = w_hh.T
    bhh = jnp.broadcast_to(b_hh[None, :], (8, 3 * H))

    h = node_states
    prop = _mm0(h, wT_msg, bmsg)
    for t in range(TIMESTEPS):
        part = edge_k(prop, srcp, tgtp, zrows)
        if t + 1 < TIMESTEPS:
            h, prop = _step(part, cnt_part, h, wT_ih, bih, wT_hh, bhh,
                            wT_msg, bmsg)
        else:
            h = _last(part, cnt_part, h, wT_ih, bih, wT_hh, bhh)
    return h, node_states


# speedup vs baseline: 1.1365x; 1.1094x over previous
import functools

import jax
import jax.numpy as jnp
from jax import lax
from jax.experimental import pallas as pl
from jax.experimental.pallas import tpu as pltpu
from jax.experimental.pallas import tpu_sc as plsc

N = 10000
H = 128
E = 320000
TIMESTEPS = 4

NC = 2
NS = 16
NW = NC * NS
CHUNK = 128
CH_PER_TILE = 80
BLK = 40
NBLK = CH_PER_TILE // BLK
E_PAD = NW * CH_PER_TILE * CHUNK
N_PAD = 10240
ROWS_PER_TILE = N_PAD // NS

RB = 400
GRID = N // RB


def _edge_body(prop_hbm, src_hbm, tgt_hbm, zr_hbm, out_hbm, srca, tgta,
               rows0, rows1, acc, g0, g1, s0, s1):
    c = lax.axis_index("c")
    s = lax.axis_index("s")
    wid = s * NC + c
    row0 = s * ROWS_PER_TILE
    pltpu.sync_copy(zr_hbm, acc.at[pl.ds(row0, ROWS_PER_TILE)])
    plsc.subcore_barrier()

    bufs = [(rows0, g0, s0), (rows1, g1, s1)]

    @pl.loop(0, NBLK)
    def _blk(blk):
        pltpu.sync_copy(src_hbm.at[wid, pl.ds(blk * BLK, BLK)], srca)
        pltpu.sync_copy(tgt_hbm.at[wid, pl.ds(blk * BLK, BLK)], tgta)
        pltpu.async_copy(prop_hbm.at[srca.at[0]], rows0, g0)
        for k in range(BLK):
            rb, gs, ss = bufs[k % 2]
            pltpu.make_async_copy(prop_hbm.at[srca.at[k]], rb, gs).wait()
            pltpu.async_copy(rb, acc.at[tgta.at[k]], ss, add=True)
            if k + 1 < BLK:
                ob, og, osem = bufs[(k + 1) % 2]
                if k >= 1:
                    pltpu.make_async_copy(ob, acc.at[tgta.at[k - 1]],
                                          osem).wait()
                pltpu.async_copy(prop_hbm.at[srca.at[k + 1]], ob, og)
        rb, _, ss = bufs[(BLK - 2) % 2]
        pltpu.make_async_copy(rb, acc.at[tgta.at[BLK - 2]], ss).wait()
        rb, _, ss = bufs[(BLK - 1) % 2]
        pltpu.make_async_copy(rb, acc.at[tgta.at[BLK - 1]], ss).wait()

    plsc.subcore_barrier()
    pltpu.sync_copy(acc.at[pl.ds(row0, ROWS_PER_TILE)],
                    out_hbm.at[c, pl.ds(row0, ROWS_PER_TILE)])


def _cnt_body(tgt_hbm, ones_hbm, z1_hbm, out_hbm, tgta, ones, acc1, csem):
    c = lax.axis_index("c")
    s = lax.axis_index("s")
    wid = s * NC + c
    row0 = s * ROWS_PER_TILE
    pltpu.sync_copy(z1_hbm, acc1.at[pl.ds(row0, ROWS_PER_TILE)])
    pltpu.sync_copy(tgt_hbm.at[wid], tgta)
    pltpu.sync_copy(ones_hbm, ones)
    plsc.subcore_barrier()

    for i in range(CH_PER_TILE):
        pltpu.async_copy(ones, acc1.at[tgta.at[i]], csem, add=True)
    for i in range(CH_PER_TILE):
        pltpu.make_async_copy(ones, acc1.at[tgta.at[i]], csem).wait()
    plsc.subcore_barrier()
    pltpu.sync_copy(acc1.at[pl.ds(row0, ROWS_PER_TILE)],
                    out_hbm.at[c, pl.ds(row0, ROWS_PER_TILE)])


@functools.lru_cache(maxsize=None)
def _sc_kernels():
    mesh = plsc.VectorSubcoreMesh(core_axis_name="c", subcore_axis_name="s",
                                  num_cores=NC, num_subcores=NS)
    edge = pl.kernel(
        _edge_body,
        out_type=jax.ShapeDtypeStruct((NC, N_PAD, H), jnp.float32),
        mesh=mesh,
        scratch_types=[
            pltpu.VMEM((BLK, CHUNK), jnp.int32),
            pltpu.VMEM((BLK, CHUNK), jnp.int32),
            pltpu.VMEM((CHUNK, H), jnp.float32),
            pltpu.VMEM((CHUNK, H), jnp.float32),
            pltpu.VMEM_SHARED((N_PAD, H), jnp.float32),
            pltpu.SemaphoreType.DMA,
            pltpu.SemaphoreType.DMA,
            pltpu.SemaphoreType.DMA,
            pltpu.SemaphoreType.DMA,
        ],
    )
    cnt = pl.kernel(
        _cnt_body,
        out_type=jax.ShapeDtypeStruct((NC, N_PAD), jnp.float32),
        mesh=mesh,
        scratch_types=[
            pltpu.VMEM((CH_PER_TILE, CHUNK), jnp.int32),
            pltpu.VMEM((CHUNK,), jnp.float32),
            pltpu.VMEM_SHARED((N_PAD,), jnp.float32),
            pltpu.SemaphoreType.DMA,
        ],
    )
    return edge, cnt


def _gru_math(p_ref, cnt_ref, h, wT_ih_ref, bih_ref, wT_hh_ref, bhh_ref):
    p = p_ref[0] + p_ref[1]
    cnt = cnt_ref[0] + cnt_ref[1]
    x = p / jnp.maximum(cnt, 1.0)
    gi = jnp.dot(x, wT_ih_ref[...], preferred_element_type=jnp.float32)
    gi = gi + bih_ref[0:1, :]
    gh = jnp.dot(h, wT_hh_ref[...], preferred_element_type=jnp.float32)
    gh = gh + bhh_ref[0:1, :]
    r = jax.nn.sigmoid(gi[:, :H] + gh[:, :H])
    z = jax.nn.sigmoid(gi[:, H:2 * H] + gh[:, H:2 * H])
    n = jnp.tanh(gi[:, 2 * H:] + r * gh[:, 2 * H:])
    return (1.0 - z) * n + z * h


def _mm0_body(h_ref, wT_msg_ref, bmsg_ref, prop_ref):
    y = jnp.dot(h_ref[...], wT_msg_ref[...],
                preferred_element_type=jnp.float32)
    prop_ref[...] = y + bmsg_ref[0:1, :]


def _step_body(p_ref, cnt_ref, h_ref, wT_ih_ref, bih_ref, wT_hh_ref,
               bhh_ref, wT_msg_ref, bmsg_ref, hn_ref, prop_ref):
    hn = _gru_math(p_ref, cnt_ref, h_ref[...], wT_ih_ref, bih_ref,
                   wT_hh_ref, bhh_ref)
    hn_ref[...] = hn
    y = jnp.dot(hn, wT_msg_ref[...], preferred_element_type=jnp.float32)
    prop_ref[...] = y + bmsg_ref[0:1, :]


def _last_body(p_ref, cnt_ref, h_ref, wT_ih_ref, bih_ref, wT_hh_ref,
               bhh_ref, hn_ref):
    hn_ref[...] = _gru_math(p_ref, cnt_ref, h_ref[...], wT_ih_ref, bih_ref,
                            wT_hh_ref, bhh_ref)


_spec_part = pl.BlockSpec((NC, RB, H), lambda i: (0, i, 0))
_spec_cnt = pl.BlockSpec((NC, RB, 1), lambda i: (0, i, 0))
_spec_h = pl.BlockSpec((RB, H), lambda i: (i, 0))
_spec_w3 = pl.BlockSpec((H, 3 * H), lambda i: (0, 0))
_spec_b3 = pl.BlockSpec((8, 3 * H), lambda i: (0, 0))
_spec_w1 = pl.BlockSpec((H, H), lambda i: (0, 0))
_spec_b1 = pl.BlockSpec((8, H), lambda i: (0, 0))

_sds_h = jax.ShapeDtypeStruct((N, H), jnp.float32)

_mm0 = pl.pallas_call(
    _mm0_body,
    grid=(GRID,),
    in_specs=[_spec_h, _spec_w1, _spec_b1],
    out_specs=_spec_h,
    out_shape=_sds_h,
)

_step = pl.pallas_call(
    _step_body,
    grid=(GRID,),
    in_specs=[_spec_part, _spec_cnt, _spec_h, _spec_w3, _spec_b3, _spec_w3,
              _spec_b3, _spec_w1, _spec_b1],
    out_specs=[_spec_h, _spec_h],
    out_shape=[_sds_h, _sds_h],
)

_last = pl.pallas_call(
    _last_body,
    grid=(GRID,),
    in_specs=[_spec_part, _spec_cnt, _spec_h, _spec_w3, _spec_b3, _spec_w3,
              _spec_b3],
    out_specs=_spec_h,
    out_shape=_sds_h,
)


def kernel(node_states, edge_lists, W_msg, b_msg, w_ih, w_hh, b_ih, b_hh):
    f32 = jnp.float32
    el = edge_lists[0]
    src = el[:, 0].astype(jnp.int32)
    tgt = el[:, 1].astype(jnp.int32)
    pad = E_PAD - E
    pad_src = jnp.arange(pad, dtype=jnp.int32) % N
    pad_tgt = N + (jnp.arange(pad, dtype=jnp.int32) % (N_PAD - N))
    srcp = jnp.concatenate([src, pad_src]).reshape(NW, CH_PER_TILE, CHUNK)
    tgtp = jnp.concatenate([tgt, pad_tgt]).reshape(NW, CH_PER_TILE, CHUNK)

    zrows = jnp.zeros((ROWS_PER_TILE, H), f32)
    z1 = jnp.zeros((ROWS_PER_TILE,), f32)
    ones1 = jnp.ones((CHUNK,), f32)

    edge_k, cnt_k = _sc_kernels()
    cnt_part = cnt_k(tgtp, ones1, z1).reshape(NC, N_PAD, 1)

    wT_msg = W_msg.T
    bmsg = jnp.broadcast_to(b_msg[None, :], (8, H))
    wT_ih = w_ih.T
    bih = jnp.broadcast_to(b_ih[None, :], (8, 3 * H))
    wT_hh = w_hh.T
    bhh = jnp.broadcast_to(b_hh[None, :], (8, 3 * H))

    h = node_states
    prop = _mm0(h, wT_msg, bmsg)
    for t in range(TIMESTEPS):
        part = edge_k(prop, srcp, tgtp, zrows)
        if t + 1 < TIMESTEPS:
            h, prop = _step(part, cnt_part, h, wT_ih, bih, wT_hh, bhh,
                            wT_msg, bmsg)
        else:
            h = _last(part, cnt_part, h, wT_ih, bih, wT_hh, bhh)
    return h, node_states


# speedup vs baseline: 1.2042x; 1.0596x over previous
import functools

import jax
import jax.numpy as jnp
from jax import lax
from jax.experimental import pallas as pl
from jax.experimental.pallas import tpu as pltpu
from jax.experimental.pallas import tpu_sc as plsc

N = 10000
H = 128
E = 320000
TIMESTEPS = 4

NC = 2
NS = 16
NW = NC * NS
CHUNK = 128
CH_PER_TILE = 80
BLK = 40
NBLK = CH_PER_TILE // BLK
E_PAD = NW * CH_PER_TILE * CHUNK
N_PAD = 10240
ROWS_PER_TILE = N_PAD // NS

RB = 1000
GRID = N // RB


def _edge_body(prop_hbm, src_hbm, tgt_hbm, zr_hbm, out_hbm, srca, tgta,
               rows0, rows1, acc, g0, g1, s0, s1):
    c = lax.axis_index("c")
    s = lax.axis_index("s")
    wid = s * NC + c
    row0 = s * ROWS_PER_TILE
    pltpu.sync_copy(zr_hbm, acc.at[pl.ds(row0, ROWS_PER_TILE)])
    plsc.subcore_barrier()

    bufs = [(rows0, g0, s0), (rows1, g1, s1)]

    @pl.loop(0, NBLK)
    def _blk(blk):
        pltpu.sync_copy(src_hbm.at[wid, pl.ds(blk * BLK, BLK)], srca)
        pltpu.sync_copy(tgt_hbm.at[wid, pl.ds(blk * BLK, BLK)], tgta)
        pltpu.async_copy(prop_hbm.at[srca.at[0]], rows0, g0)
        for k in range(BLK):
            rb, gs, ss = bufs[k % 2]
            pltpu.make_async_copy(prop_hbm.at[srca.at[k]], rb, gs).wait()
            pltpu.async_copy(rb, acc.at[tgta.at[k]], ss, add=True)
            if k + 1 < BLK:
                ob, og, osem = bufs[(k + 1) % 2]
                if k >= 1:
                    pltpu.make_async_copy(ob, acc.at[tgta.at[k - 1]],
                                          osem).wait()
                pltpu.async_copy(prop_hbm.at[srca.at[k + 1]], ob, og)
        rb, _, ss = bufs[(BLK - 2) % 2]
        pltpu.make_async_copy(rb, acc.at[tgta.at[BLK - 2]], ss).wait()
        rb, _, ss = bufs[(BLK - 1) % 2]
        pltpu.make_async_copy(rb, acc.at[tgta.at[BLK - 1]], ss).wait()

    plsc.subcore_barrier()
    pltpu.sync_copy(acc.at[pl.ds(row0, ROWS_PER_TILE)],
                    out_hbm.at[c, pl.ds(row0, ROWS_PER_TILE)])


def _cnt_body(tgt_hbm, ones_hbm, z1_hbm, out_hbm, tgta, ones, acc1, csem):
    c = lax.axis_index("c")
    s = lax.axis_index("s")
    wid = s * NC + c
    row0 = s * ROWS_PER_TILE
    pltpu.sync_copy(z1_hbm, acc1.at[pl.ds(row0, ROWS_PER_TILE)])
    pltpu.sync_copy(tgt_hbm.at[wid], tgta)
    pltpu.sync_copy(ones_hbm, ones)
    plsc.subcore_barrier()

    for i in range(CH_PER_TILE):
        pltpu.async_copy(ones, acc1.at[tgta.at[i]], csem, add=True)
    for i in range(CH_PER_TILE):
        pltpu.make_async_copy(ones, acc1.at[tgta.at[i]], csem).wait()
    plsc.subcore_barrier()
    pltpu.sync_copy(acc1.at[pl.ds(row0, ROWS_PER_TILE)],
                    out_hbm.at[c, pl.ds(row0, ROWS_PER_TILE)])


@functools.lru_cache(maxsize=None)
def _sc_kernels():
    mesh = plsc.VectorSubcoreMesh(core_axis_name="c", subcore_axis_name="s",
                                  num_cores=NC, num_subcores=NS)
    edge = pl.kernel(
        _edge_body,
        out_type=jax.ShapeDtypeStruct((NC, N_PAD, H), jnp.float32),
        mesh=mesh,
        scratch_types=[
            pltpu.VMEM((BLK, CHUNK), jnp.int32),
            pltpu.VMEM((BLK, CHUNK), jnp.int32),
            pltpu.VMEM((CHUNK, H), jnp.float32),
            pltpu.VMEM((CHUNK, H), jnp.float32),
            pltpu.VMEM_SHARED((N_PAD, H), jnp.float32),
            pltpu.SemaphoreType.DMA,
            pltpu.SemaphoreType.DMA,
            pltpu.SemaphoreType.DMA,
            pltpu.SemaphoreType.DMA,
        ],
    )
    cnt = pl.kernel(
        _cnt_body,
        out_type=jax.ShapeDtypeStruct((NC, N_PAD), jnp.float32),
        mesh=mesh,
        scratch_types=[
            pltpu.VMEM((CH_PER_TILE, CHUNK), jnp.int32),
            pltpu.VMEM((CHUNK,), jnp.float32),
            pltpu.VMEM_SHARED((N_PAD,), jnp.float32),
            pltpu.SemaphoreType.DMA,
        ],
    )
    return edge, cnt


def _gru_math(p_ref, cnt_ref, h, wT_ih_ref, bih_ref, wT_hh_ref, bhh_ref):
    p = p_ref[0] + p_ref[1]
    cnt = cnt_ref[0] + cnt_ref[1]
    x = p / jnp.maximum(cnt, 1.0)
    gi = jnp.dot(x, wT_ih_ref[...], preferred_element_type=jnp.float32)
    gi = gi + bih_ref[0:1, :]
    gh = jnp.dot(h, wT_hh_ref[...], preferred_element_type=jnp.float32)
    gh = gh + bhh_ref[0:1, :]
    r = jax.nn.sigmoid(gi[:, :H] + gh[:, :H])
    z = jax.nn.sigmoid(gi[:, H:2 * H] + gh[:, H:2 * H])
    n = jnp.tanh(gi[:, 2 * H:] + r * gh[:, 2 * H:])
    return (1.0 - z) * n + z * h


def _mm0_body(h_ref, wT_msg_ref, bmsg_ref, prop_ref):
    y = jnp.dot(h_ref[...], wT_msg_ref[...],
                preferred_element_type=jnp.float32)
    prop_ref[...] = y + bmsg_ref[0:1, :]


def _step_body(p_ref, cnt_ref, h_ref, wT_ih_ref, bih_ref, wT_hh_ref,
               bhh_ref, wT_msg_ref, bmsg_ref, hn_ref, prop_ref):
    hn = _gru_math(p_ref, cnt_ref, h_ref[...], wT_ih_ref, bih_ref,
                   wT_hh_ref, bhh_ref)
    hn_ref[...] = hn
    y = jnp.dot(hn, wT_msg_ref[...], preferred_element_type=jnp.float32)
    prop_ref[...] = y + bmsg_ref[0:1, :]


def _last_body(p_ref, cnt_ref, h_ref, wT_ih_ref, bih_ref, wT_hh_ref,
               bhh_ref, hn_ref):
    hn_ref[...] = _gru_math(p_ref, cnt_ref, h_ref[...], wT_ih_ref, bih_ref,
                            wT_hh_ref, bhh_ref)


_spec_part = pl.BlockSpec((NC, RB, H), lambda i: (0, i, 0))
_spec_cnt = pl.BlockSpec((NC, RB, 1), lambda i: (0, i, 0))
_spec_h = pl.BlockSpec((RB, H), lambda i: (i, 0))
_spec_w3 = pl.BlockSpec((H, 3 * H), lambda i: (0, 0))
_spec_b3 = pl.BlockSpec((8, 3 * H), lambda i: (0, 0))
_spec_w1 = pl.BlockSpec((H, H), lambda i: (0, 0))
_spec_b1 = pl.BlockSpec((8, H), lambda i: (0, 0))

_sds_h = jax.ShapeDtypeStruct((N, H), jnp.float32)

_mm0 = pl.pallas_call(
    _mm0_body,
    grid=(GRID,),
    in_specs=[_spec_h, _spec_w1, _spec_b1],
    out_specs=_spec_h,
    out_shape=_sds_h,
)

_step = pl.pallas_call(
    _step_body,
    grid=(GRID,),
    in_specs=[_spec_part, _spec_cnt, _spec_h, _spec_w3, _spec_b3, _spec_w3,
              _spec_b3, _spec_w1, _spec_b1],
    out_specs=[_spec_h, _spec_h],
    out_shape=[_sds_h, _sds_h],
)

_last = pl.pallas_call(
    _last_body,
    grid=(GRID,),
    in_specs=[_spec_part, _spec_cnt, _spec_h, _spec_w3, _spec_b3, _spec_w3,
              _spec_b3],
    out_specs=_spec_h,
    out_shape=_sds_h,
)


def kernel(node_states, edge_lists, W_msg, b_msg, w_ih, w_hh, b_ih, b_hh):
    f32 = jnp.float32
    el = edge_lists[0]
    src = el[:, 0].astype(jnp.int32)
    tgt = el[:, 1].astype(jnp.int32)
    pad = E_PAD - E
    pad_src = jnp.arange(pad, dtype=jnp.int32) % N
    pad_tgt = N + (jnp.arange(pad, dtype=jnp.int32) % (N_PAD - N))
    srcp = jnp.concatenate([src, pad_src]).reshape(NW, CH_PER_TILE, CHUNK)
    tgtp = jnp.concatenate([tgt, pad_tgt]).reshape(NW, CH_PER_TILE, CHUNK)

    zrows = jnp.zeros((ROWS_PER_TILE, H), f32)
    z1 = jnp.zeros((ROWS_PER_TILE,), f32)
    ones1 = jnp.ones((CHUNK,), f32)

    edge_k, cnt_k = _sc_kernels()
    cnt_part = cnt_k(tgtp, ones1, z1).reshape(NC, N_PAD, 1)

    wT_msg = W_msg.T
    bmsg = jnp.broadcast_to(b_msg[None, :], (8, H))
    wT_ih = w_ih.T
    bih = jnp.broadcast_to(b_ih[None, :], (8, 3 * H))
    wT_hh = w_hh.T
    bhh = jnp.broadcast_to(b_hh[None, :], (8, 3 * H))

    h = node_states
    prop = _mm0(h, wT_msg, bmsg)
    for t in range(TIMESTEPS):
        part = edge_k(prop, srcp, tgtp, zrows)
        if t + 1 < TIMESTEPS:
            h, prop = _step(part, cnt_part, h, wT_ih, bih, wT_hh, bhh,
                            wT_msg, bmsg)
        else:
            h = _last(part, cnt_part, h, wT_ih, bih, wT_hh, bhh)
    return h, node_states


# speedup vs baseline: 1.2225x; 1.0152x over previous
import functools

import jax
import jax.numpy as jnp
from jax import lax
from jax.experimental import pallas as pl
from jax.experimental.pallas import tpu as pltpu
from jax.experimental.pallas import tpu_sc as plsc

N = 10000
H = 128
E = 320000
TIMESTEPS = 4

NC = 2
NS = 16
NW = NC * NS
CHUNK = 128
CH_PER_TILE = 80
BLK = 40
NBLK = CH_PER_TILE // BLK
E_PAD = NW * CH_PER_TILE * CHUNK
N_PAD = 10240
ROWS_PER_TILE = N_PAD // NS

RB = 2000
GRID = N // RB


def _edge_body(prop_hbm, src_hbm, tgt_hbm, zr_hbm, out_hbm, srca, tgta,
               rows0, rows1, acc, g0, g1, s0, s1):
    c = lax.axis_index("c")
    s = lax.axis_index("s")
    wid = s * NC + c
    row0 = s * ROWS_PER_TILE
    pltpu.sync_copy(zr_hbm, acc.at[pl.ds(row0, ROWS_PER_TILE)])
    plsc.subcore_barrier()

    bufs = [(rows0, g0, s0), (rows1, g1, s1)]

    @pl.loop(0, NBLK)
    def _blk(blk):
        pltpu.sync_copy(src_hbm.at[wid, pl.ds(blk * BLK, BLK)], srca)
        pltpu.sync_copy(tgt_hbm.at[wid, pl.ds(blk * BLK, BLK)], tgta)
        pltpu.async_copy(prop_hbm.at[srca.at[0]], rows0, g0)
        for k in range(BLK):
            rb, gs, ss = bufs[k % 2]
            pltpu.make_async_copy(prop_hbm.at[srca.at[k]], rb, gs).wait()
            pltpu.async_copy(rb, acc.at[tgta.at[k]], ss, add=True)
            if k + 1 < BLK:
                ob, og, osem = bufs[(k + 1) % 2]
                if k >= 1:
                    pltpu.make_async_copy(ob, acc.at[tgta.at[k - 1]],
                                          osem).wait()
                pltpu.async_copy(prop_hbm.at[srca.at[k + 1]], ob, og)
        rb, _, ss = bufs[(BLK - 2) % 2]
        pltpu.make_async_copy(rb, acc.at[tgta.at[BLK - 2]], ss).wait()
        rb, _, ss = bufs[(BLK - 1) % 2]
        pltpu.make_async_copy(rb, acc.at[tgta.at[BLK - 1]], ss).wait()

    plsc.subcore_barrier()
    pltpu.sync_copy(acc.at[pl.ds(row0, ROWS_PER_TILE)],
                    out_hbm.at[c, pl.ds(row0, ROWS_PER_TILE)])


def _cnt_body(tgt_hbm, ones_hbm, z1_hbm, out_hbm, tgta, ones, acc1, csem):
    c = lax.axis_index("c")
    s = lax.axis_index("s")
    wid = s * NC + c
    row0 = s * ROWS_PER_TILE
    pltpu.sync_copy(z1_hbm, acc1.at[pl.ds(row0, ROWS_PER_TILE)])
    pltpu.sync_copy(tgt_hbm.at[wid], tgta)
    pltpu.sync_copy(ones_hbm, ones)
    plsc.subcore_barrier()

    for i in range(CH_PER_TILE):
        pltpu.async_copy(ones, acc1.at[tgta.at[i]], csem, add=True)
    for i in range(CH_PER_TILE):
        pltpu.make_async_copy(ones, acc1.at[tgta.at[i]], csem).wait()
    plsc.subcore_barrier()
    pltpu.sync_copy(acc1.at[pl.ds(row0, ROWS_PER_TILE)],
                    out_hbm.at[c, pl.ds(row0, ROWS_PER_TILE)])


@functools.lru_cache(maxsize=None)
def _sc_kernels():
    mesh = plsc.VectorSubcoreMesh(core_axis_name="c", subcore_axis_name="s",
                                  num_cores=NC, num_subcores=NS)
    edge = pl.kernel(
        _edge_body,
        out_type=jax.ShapeDtypeStruct((NC, N_PAD, H), jnp.float32),
        mesh=mesh,
        scratch_types=[
            pltpu.VMEM((BLK, CHUNK), jnp.int32),
            pltpu.VMEM((BLK, CHUNK), jnp.int32),
            pltpu.VMEM((CHUNK, H), jnp.float32),
            pltpu.VMEM((CHUNK, H), jnp.float32),
            pltpu.VMEM_SHARED((N_PAD, H), jnp.float32),
            pltpu.SemaphoreType.DMA,
            pltpu.SemaphoreType.DMA,
            pltpu.SemaphoreType.DMA,
            pltpu.SemaphoreType.DMA,
        ],
    )
    cnt = pl.kernel(
        _cnt_body,
        out_type=jax.ShapeDtypeStruct((NC, N_PAD), jnp.float32),
        mesh=mesh,
        scratch_types=[
            pltpu.VMEM((CH_PER_TILE, CHUNK), jnp.int32),
            pltpu.VMEM((CHUNK,), jnp.float32),
            pltpu.VMEM_SHARED((N_PAD,), jnp.float32),
            pltpu.SemaphoreType.DMA,
        ],
    )
    return edge, cnt


def _gru_math(p_ref, cnt_ref, h, wT_ih_ref, bih_ref, wT_hh_ref, bhh_ref):
    p = p_ref[0] + p_ref[1]
    cnt = cnt_ref[0] + cnt_ref[1]
    x = p / jnp.maximum(cnt, 1.0)
    gi = jnp.dot(x, wT_ih_ref[...], preferred_element_type=jnp.float32)
    gi = gi + bih_ref[0:1, :]
    gh = jnp.dot(h, wT_hh_ref[...], preferred_element_type=jnp.float32)
    gh = gh + bhh_ref[0:1, :]
    r = jax.nn.sigmoid(gi[:, :H] + gh[:, :H])
    z = jax.nn.sigmoid(gi[:, H:2 * H] + gh[:, H:2 * H])
    n = jnp.tanh(gi[:, 2 * H:] + r * gh[:, 2 * H:])
    return (1.0 - z) * n + z * h


def _mm0_body(h_ref, wT_msg_ref, bmsg_ref, prop_ref):
    y = jnp.dot(h_ref[...], wT_msg_ref[...],
                preferred_element_type=jnp.float32)
    prop_ref[...] = y + bmsg_ref[0:1, :]


def _step_body(p_ref, cnt_ref, h_ref, wT_ih_ref, bih_ref, wT_hh_ref,
               bhh_ref, wT_msg_ref, bmsg_ref, hn_ref, prop_ref):
    hn = _gru_math(p_ref, cnt_ref, h_ref[...], wT_ih_ref, bih_ref,
                   wT_hh_ref, bhh_ref)
    hn_ref[...] = hn
    y = jnp.dot(hn, wT_msg_ref[...], preferred_element_type=jnp.float32)
    prop_ref[...] = y + bmsg_ref[0:1, :]


def _last_body(p_ref, cnt_ref, h_ref, wT_ih_ref, bih_ref, wT_hh_ref,
               bhh_ref, hn_ref):
    hn_ref[...] = _gru_math(p_ref, cnt_ref, h_ref[...], wT_ih_ref, bih_ref,
                            wT_hh_ref, bhh_ref)


_spec_part = pl.BlockSpec((NC, RB, H), lambda i: (0, i, 0))
_spec_cnt = pl.BlockSpec((NC, RB, 1), lambda i: (0, i, 0))
_spec_h = pl.BlockSpec((RB, H), lambda i: (i, 0))
_spec_w3 = pl.BlockSpec((H, 3 * H), lambda i: (0, 0))
_spec_b3 = pl.BlockSpec((8, 3 * H), lambda i: (0, 0))
_spec_w1 = pl.BlockSpec((H, H), lambda i: (0, 0))
_spec_b1 = pl.BlockSpec((8, H), lambda i: (0, 0))

_sds_h = jax.ShapeDtypeStruct((N, H), jnp.float32)

_mm0 = pl.pallas_call(
    _mm0_body,
    grid=(GRID,),
    in_specs=[_spec_h, _spec_w1, _spec_b1],
    out_specs=_spec_h,
    out_shape=_sds_h,
)

_step = pl.pallas_call(
    _step_body,
    grid=(GRID,),
    in_specs=[_spec_part, _spec_cnt, _spec_h, _spec_w3, _spec_b3, _spec_w3,
              _spec_b3, _spec_w1, _spec_b1],
    out_specs=[_spec_h, _spec_h],
    out_shape=[_sds_h, _sds_h],
)

_last = pl.pallas_call(
    _last_body,
    grid=(GRID,),
    in_specs=[_spec_part, _spec_cnt, _spec_h, _spec_w3, _spec_b3, _spec_w3,
              _spec_b3],
    out_specs=_spec_h,
    out_shape=_sds_h,
)


def kernel(node_states, edge_lists, W_msg, b_msg, w_ih, w_hh, b_ih, b_hh):
    f32 = jnp.float32
    el = edge_lists[0]
    src = el[:, 0].astype(jnp.int32)
    tgt = el[:, 1].astype(jnp.int32)
    pad = E_PAD - E
    pad_src = jnp.arange(pad, dtype=jnp.int32) % N
    pad_tgt = N + (jnp.arange(pad, dtype=jnp.int32) % (N_PAD - N))
    srcp = jnp.concatenate([src, pad_src]).reshape(NW, CH_PER_TILE, CHUNK)
    tgtp = jnp.concatenate([tgt, pad_tgt]).reshape(NW, CH_PER_TILE, CHUNK)

    zrows = jnp.zeros((ROWS_PER_TILE, H), f32)
    z1 = jnp.zeros((ROWS_PER_TILE,), f32)
    ones1 = jnp.ones((CHUNK,), f32)

    edge_k, cnt_k = _sc_kernels()
    cnt_part = cnt_k(tgtp, ones1, z1).reshape(NC, N_PAD, 1)

    wT_msg = W_msg.T
    bmsg = jnp.broadcast_to(b_msg[None, :], (8, H))
    wT_ih = w_ih.T
    bih = jnp.broadcast_to(b_ih[None, :], (8, 3 * H))
    wT_hh = w_hh.T
    bhh = jnp.broadcast_to(b_hh[None, :], (8, 3 * H))

    h = node_states
    prop = _mm0(h, wT_msg, bmsg)
    for t in range(TIMESTEPS):
        part = edge_k(prop, srcp, tgtp, zrows)
        if t + 1 < TIMESTEPS:
            h, prop = _step(part, cnt_part, h, wT_ih, bih, wT_hh, bhh,
                            wT_msg, bmsg)
        else:
            h = _last(part, cnt_part, h, wT_ih, bih, wT_hh, bhh)
    return h, node_states


# speedup vs baseline: 1.2364x; 1.0114x over previous
import functools

import jax
import jax.numpy as jnp
from jax import lax
from jax.experimental import pallas as pl
from jax.experimental.pallas import tpu as pltpu
from jax.experimental.pallas import tpu_sc as plsc

N = 10000
H = 128
E = 320000
TIMESTEPS = 4

NC = 2
NS = 16
NW = NC * NS
CHUNK = 128
CH_PER_TILE = 80
BLK = 16
NBLK = CH_PER_TILE // BLK
E_PAD = NW * CH_PER_TILE * CHUNK
N_PAD = 10240
ROWS_PER_TILE = N_PAD // NS

RB = 2000
GRID = N // RB


def _edge_body(prop_hbm, src_hbm, tgt_hbm, zr_hbm, out_hbm, srca0, tgta0,
               srca1, tgta1, rows0, rows1, acc, g0, g1, s0, s1, isem):
    c = lax.axis_index("c")
    s = lax.axis_index("s")
    wid = s * NC + c
    row0 = s * ROWS_PER_TILE

    ib = [(srca0, tgta0), (srca1, tgta1)]
    bufs = [(rows0, g0, s0), (rows1, g1, s1)]

    def sidx(blk):
        return (src_hbm.at[wid, pl.ds(blk * BLK, BLK)],
                tgt_hbm.at[wid, pl.ds(blk * BLK, BLK)])

    def cidx(k):
        sb, tb = ib[(k // BLK) % 2]
        return sb.at[k % BLK], tb.at[k % BLK]

    sh, th = sidx(0)
    pltpu.sync_copy(sh, srca0)
    pltpu.sync_copy(th, tgta0)
    pltpu.async_copy(prop_hbm.at[srca0.at[0]], rows0, g0)
    pltpu.sync_copy(zr_hbm, acc.at[pl.ds(row0, ROWS_PER_TILE)])
    plsc.subcore_barrier()

    for k in range(CH_PER_TILE):
        blk, pos = k // BLK, k % BLK
        rb, gs, ss = bufs[k % 2]
        sr, tr = cidx(k)
        if pos == 1 and blk + 1 < NBLK:
            sh, th = sidx(blk + 1)
            nsb, ntb = ib[(blk + 1) % 2]
            pltpu.async_copy(sh, nsb, isem)
            pltpu.async_copy(th, ntb, isem)
        if pos == BLK - 1 and blk + 1 < NBLK:
            sh, th = sidx(blk + 1)
            nsb, ntb = ib[(blk + 1) % 2]
            pltpu.make_async_copy(sh, nsb, isem).wait()
            pltpu.make_async_copy(th, ntb, isem).wait()
        pltpu.make_async_copy(prop_hbm.at[sr], rb, gs).wait()
        pltpu.async_copy(rb, acc.at[tr], ss, add=True)
        if k + 1 < CH_PER_TILE:
            ob, og, osem = bufs[(k + 1) % 2]
            if k >= 1:
                _, ptr = cidx(k - 1)
                pltpu.make_async_copy(ob, acc.at[ptr], osem).wait()
            nsr, _ = cidx(k + 1)
            pltpu.async_copy(prop_hbm.at[nsr], ob, og)
    for k in (CH_PER_TILE - 2, CH_PER_TILE - 1):
        rb, _, ss = bufs[k % 2]
        _, tr = cidx(k)
        pltpu.make_async_copy(rb, acc.at[tr], ss).wait()

    plsc.subcore_barrier()
    pltpu.sync_copy(acc.at[pl.ds(row0, ROWS_PER_TILE)],
                    out_hbm.at[c, pl.ds(row0, ROWS_PER_TILE)])


def _cnt_body(tgt_hbm, ones_hbm, z1_hbm, out_hbm, tgta, ones, acc1, csem):
    c = lax.axis_index("c")
    s = lax.axis_index("s")
    wid = s * NC + c
    row0 = s * ROWS_PER_TILE
    pltpu.sync_copy(z1_hbm, acc1.at[pl.ds(row0, ROWS_PER_TILE)])
    pltpu.sync_copy(tgt_hbm.at[wid], tgta)
    pltpu.sync_copy(ones_hbm, ones)
    plsc.subcore_barrier()

    for i in range(CH_PER_TILE):
        pltpu.async_copy(ones, acc1.at[tgta.at[i]], csem, add=True)
    for i in range(CH_PER_TILE):
        pltpu.make_async_copy(ones, acc1.at[tgta.at[i]], csem).wait()
    plsc.subcore_barrier()
    pltpu.sync_copy(acc1.at[pl.ds(row0, ROWS_PER_TILE)],
                    out_hbm.at[c, pl.ds(row0, ROWS_PER_TILE)])


@functools.lru_cache(maxsize=None)
def _sc_kernels():
    mesh = plsc.VectorSubcoreMesh(core_axis_name="c", subcore_axis_name="s",
                                  num_cores=NC, num_subcores=NS)
    edge = pl.kernel(
        _edge_body,
        out_type=jax.ShapeDtypeStruct((NC, N_PAD, H), jnp.float32),
        mesh=mesh,
        scratch_types=[
            pltpu.VMEM((BLK, CHUNK), jnp.int32),
            pltpu.VMEM((BLK, CHUNK), jnp.int32),
            pltpu.VMEM((BLK, CHUNK), jnp.int32),
            pltpu.VMEM((BLK, CHUNK), jnp.int32),
            pltpu.VMEM((CHUNK, H), jnp.float32),
            pltpu.VMEM((CHUNK, H), jnp.float32),
            pltpu.VMEM_SHARED((N_PAD, H), jnp.float32),
            pltpu.SemaphoreType.DMA,
            pltpu.SemaphoreType.DMA,
            pltpu.SemaphoreType.DMA,
            pltpu.SemaphoreType.DMA,
            pltpu.SemaphoreType.DMA,
        ],
    )
    cnt = pl.kernel(
        _cnt_body,
        out_type=jax.ShapeDtypeStruct((NC, N_PAD), jnp.float32),
        mesh=mesh,
        scratch_types=[
            pltpu.VMEM((CH_PER_TILE, CHUNK), jnp.int32),
            pltpu.VMEM((CHUNK,), jnp.float32),
            pltpu.VMEM_SHARED((N_PAD,), jnp.float32),
            pltpu.SemaphoreType.DMA,
        ],
    )
    return edge, cnt


def _gru_math(p_ref, cnt_ref, h, wT_ih_ref, bih_ref, wT_hh_ref, bhh_ref):
    p = p_ref[0] + p_ref[1]
    cnt = cnt_ref[0] + cnt_ref[1]
    x = p / jnp.maximum(cnt, 1.0)
    gi = jnp.dot(x, wT_ih_ref[...], preferred_element_type=jnp.float32)
    gi = gi + bih_ref[0:1, :]
    gh = jnp.dot(h, wT_hh_ref[...], preferred_element_type=jnp.float32)
    gh = gh + bhh_ref[0:1, :]
    r = jax.nn.sigmoid(gi[:, :H] + gh[:, :H])
    z = jax.nn.sigmoid(gi[:, H:2 * H] + gh[:, H:2 * H])
    n = jnp.tanh(gi[:, 2 * H:] + r * gh[:, 2 * H:])
    return (1.0 - z) * n + z * h


def _mm0_body(h_ref, wT_msg_ref, bmsg_ref, prop_ref):
    y = jnp.dot(h_ref[...], wT_msg_ref[...],
                preferred_element_type=jnp.float32)
    prop_ref[...] = y + bmsg_ref[0:1, :]


def _step_body(p_ref, cnt_ref, h_ref, wT_ih_ref, bih_ref, wT_hh_ref,
               bhh_ref, wT_msg_ref, bmsg_ref, hn_ref, prop_ref):
    hn = _gru_math(p_ref, cnt_ref, h_ref[...], wT_ih_ref, bih_ref,
                   wT_hh_ref, bhh_ref)
    hn_ref[...] = hn
    y = jnp.dot(hn, wT_msg_ref[...], preferred_element_type=jnp.float32)
    prop_ref[...] = y + bmsg_ref[0:1, :]


def _last_body(p_ref, cnt_ref, h_ref, wT_ih_ref, bih_ref, wT_hh_ref,
               bhh_ref, hn_ref):
    hn_ref[...] = _gru_math(p_ref, cnt_ref, h_ref[...], wT_ih_ref, bih_ref,
                            wT_hh_ref, bhh_ref)


_spec_part = pl.BlockSpec((NC, RB, H), lambda i: (0, i, 0))
_spec_cnt = pl.BlockSpec((NC, RB, 1), lambda i: (0, i, 0))
_spec_h = pl.BlockSpec((RB, H), lambda i: (i, 0))
_spec_w3 = pl.BlockSpec((H, 3 * H), lambda i: (0, 0))
_spec_b3 = pl.BlockSpec((8, 3 * H), lambda i: (0, 0))
_spec_w1 = pl.BlockSpec((H, H), lambda i: (0, 0))
_spec_b1 = pl.BlockSpec((8, H), lambda i: (0, 0))

_sds_h = jax.ShapeDtypeStruct((N, H), jnp.float32)

_mm0 = pl.pallas_call(
    _mm0_body,
    grid=(GRID,),
    in_specs=[_spec_h, _spec_w1, _spec_b1],
    out_specs=_spec_h,
    out_shape=_sds_h,
)

_step = pl.pallas_call(
    _step_body,
    grid=(GRID,),
    in_specs=[_spec_part, _spec_cnt, _spec_h, _spec_w3, _spec_b3, _spec_w3,
              _spec_b3, _spec_w1, _spec_b1],
    out_specs=[_spec_h, _spec_h],
    out_shape=[_sds_h, _sds_h],
)

_last = pl.pallas_call(
    _last_body,
    grid=(GRID,),
    in_specs=[_spec_part, _spec_cnt, _spec_h, _spec_w3, _spec_b3, _spec_w3,
              _spec_b3],
    out_specs=_spec_h,
    out_shape=_sds_h,
)


def kernel(node_states, edge_lists, W_msg, b_msg, w_ih, w_hh, b_ih, b_hh):
    f32 = jnp.float32
    el = edge_lists[0]
    src = el[:, 0].astype(jnp.int32)
    tgt = el[:, 1].astype(jnp.int32)
    pad = E_PAD - E
    pad_src = jnp.arange(pad, dtype=jnp.int32) % N
    pad_tgt = N + (jnp.arange(pad, dtype=jnp.int32) % (N_PAD - N))
    srcp = jnp.concatenate([src, pad_src]).reshape(NW, CH_PER_TILE, CHUNK)
    tgtp = jnp.concatenate([tgt, pad_tgt]).reshape(NW, CH_PER_TILE, CHUNK)

    zrows = jnp.zeros((ROWS_PER_TILE, H), f32)
    z1 = jnp.zeros((ROWS_PER_TILE,), f32)
    ones1 = jnp.ones((CHUNK,), f32)

    edge_k, cnt_k = _sc_kernels()
    cnt_part = cnt_k(tgtp, ones1, z1).reshape(NC, N_PAD, 1)

    wT_msg = W_msg.T
    bmsg = jnp.broadcast_to(b_msg[None, :], (8, H))
    wT_ih = w_ih.T
    bih = jnp.broadcast_to(b_ih[None, :], (8, 3 * H))
    wT_hh = w_hh.T
    bhh = jnp.broadcast_to(b_hh[None, :], (8, 3 * H))

    h = node_states
    prop = _mm0(h, wT_msg, bmsg)
    for t in range(TIMESTEPS):
        part = edge_k(prop, srcp, tgtp, zrows)
        if t + 1 < TIMESTEPS:
            h, prop = _step(part, cnt_part, h, wT_ih, bih, wT_hh, bhh,
                            wT_msg, bmsg)
        else:
            h = _last(part, cnt_part, h, wT_ih, bih, wT_hh, bhh)
    return h, node_states
